# Initial kernel scaffold; baseline (speedup 1.0000x reference)
import jax
import jax.numpy as jnp
from jax.experimental import pallas as pl


def kernel(x, Wq, bq, Wk, bk, Wv, bv, gamma):
    raise NotImplementedError("write your pallas kernel here")


# speedup vs baseline: 6.6681x; 6.6681x over previous
import jax
import jax.numpy as jnp
from jax.experimental import pallas as pl

_OFFS = tuple((dr, dc) for dr in (-1, 0, 1) for dc in (-1, 0, 1))


def _roll_lanes(a, shift):
    s = shift % a.shape[-1]
    if s == 0:
        return a
    return jnp.concatenate([a[:, s:], a[:, :s]], axis=1)


def _body(x_ref, wq_ref, bq_ref, wk_ref, bk_ref, wv_ref, bv_ref, g_ref,
          out_ref, att_ref, *, width, height):
    n_pos = width * height
    xf = x_ref[0]
    q = jnp.dot(wq_ref[...], xf, preferred_element_type=jnp.float32) + bq_ref[...]
    k = jnp.dot(wk_ref[...], xf, preferred_element_type=jnp.float32) + bk_ref[...]
    v = jnp.dot(wv_ref[...], xf, preferred_element_type=jnp.float32) + bv_ref[...]

    n_iota = jax.lax.broadcasted_iota(jnp.int32, (1, n_pos), 1)
    r = n_iota // height
    c = n_iota % height

    energies = []
    for dr, dc in _OFFS:
        off = dr * height + dc
        kr = _roll_lanes(k, off)
        e = jnp.sum(q * kr, axis=0, keepdims=True)
        valid = ((r + dr >= 0) & (r + dr < width)
                 & (c + dc >= 0) & (c + dc < height))
        energies.append(jnp.where(valid, e, -1e30))
    energy = jnp.concatenate(energies, axis=0)
    emax = jnp.max(energy, axis=0, keepdims=True)
    p = jnp.exp(energy - emax)
    attn = p / jnp.sum(p, axis=0, keepdims=True)

    acc = jnp.zeros_like(v)
    for i, (dr, dc) in enumerate(_OFFS):
        off = dr * height + dc
        acc = acc + attn[i:i + 1, :] * _roll_lanes(v, off)
    out_ref[0] = g_ref[0, 0] * acc + xf

    row_i = jax.lax.broadcasted_iota(jnp.int32, (n_pos, n_pos), 0)
    col_i = jax.lax.broadcasted_iota(jnp.int32, (n_pos, n_pos), 1)
    diff = col_i - row_i
    band = jnp.zeros((n_pos, n_pos), jnp.float32)
    for i, (dr, dc) in enumerate(_OFFS):
        off = dr * height + dc
        rolled = _roll_lanes(attn[i:i + 1, :], -off)
        band = jnp.where(diff == off, rolled, band)
    att_ref[0] = band


def kernel(x, Wq, bq, Wk, bk, Wv, bv, gamma):
    B, C, width, height = x.shape
    N = width * height
    d = Wq.shape[0]
    xf = x.reshape(B, C, N)

    import functools
    body = functools.partial(_body, width=width, height=height)
    out3, att = pl.pallas_call(
        body,
        grid=(B,),
        in_specs=[
            pl.BlockSpec((1, C, N), lambda b: (b, 0, 0)),
            pl.BlockSpec((d, C), lambda b: (0, 0)),
            pl.BlockSpec((d, 1), lambda b: (0, 0)),
            pl.BlockSpec((d, C), lambda b: (0, 0)),
            pl.BlockSpec((d, 1), lambda b: (0, 0)),
            pl.BlockSpec((C, C), lambda b: (0, 0)),
            pl.BlockSpec((C, 1), lambda b: (0, 0)),
            pl.BlockSpec((1, 1), lambda b: (0, 0)),
        ],
        out_specs=[
            pl.BlockSpec((1, C, N), lambda b: (b, 0, 0)),
            pl.BlockSpec((1, N, N), lambda b: (b, 0, 0)),
        ],
        out_shape=[
            jax.ShapeDtypeStruct((B, C, N), jnp.float32),
            jax.ShapeDtypeStruct((B, N, N), jnp.float32),
        ],
    )(xf, Wq, bq.reshape(d, 1), Wk, bk.reshape(d, 1), Wv,
      bv.reshape(C, 1), gamma.reshape(1, 1))
    return out3.reshape(B, C, width, height), att


# speedup vs baseline: 6.7254x; 1.0086x over previous
import jax
import jax.numpy as jnp
from jax.experimental import pallas as pl

_OFFS = tuple((dr, dc) for dr in (-1, 0, 1) for dc in (-1, 0, 1))


def _roll_lanes(a, shift):
    s = shift % a.shape[-1]
    if s == 0:
        return a
    return jnp.concatenate([a[:, s:], a[:, :s]], axis=1)


def _body(x_ref, wq_ref, bq_ref, wk_ref, bk_ref, wv_ref, bv_ref, g_ref,
          out_ref, att_ref, *, width, height):
    n_pos = width * height
    xf = x_ref[0]
    q = jnp.dot(wq_ref[...], xf, preferred_element_type=jnp.float32) + bq_ref[...]
    k = jnp.dot(wk_ref[...], xf, preferred_element_type=jnp.float32) + bk_ref[...]
    v = jnp.dot(wv_ref[...], xf, preferred_element_type=jnp.float32) + bv_ref[...]

    n_iota = jax.lax.broadcasted_iota(jnp.int32, (1, n_pos), 1)
    r = n_iota // height
    c = n_iota % height

    energies = []
    for dr, dc in _OFFS:
        off = dr * height + dc
        kr = _roll_lanes(k, off)
        e = jnp.sum(q * kr, axis=0, keepdims=True)
        valid = ((r + dr >= 0) & (r + dr < width)
                 & (c + dc >= 0) & (c + dc < height))
        energies.append(jnp.where(valid, e, -1e30))
    energy = jnp.concatenate(energies, axis=0)
    emax = jnp.max(energy, axis=0, keepdims=True)
    p = jnp.exp(energy - emax)
    attn = p / jnp.sum(p, axis=0, keepdims=True)

    acc = jnp.zeros_like(v)
    for i, (dr, dc) in enumerate(_OFFS):
        off = dr * height + dc
        acc = acc + attn[i:i + 1, :] * _roll_lanes(v, off)
    out_ref[0] = g_ref[0, 0] * acc + xf

    att_ref[0] = jnp.zeros((n_pos, n_pos), jnp.float32)
    rolled = []
    for i, (dr, dc) in enumerate(_OFFS):
        off = dr * height + dc
        rolled.append(_roll_lanes(attn[i:i + 1, :], -off))
    CH, CW = 128, 384
    row_b = jax.lax.broadcasted_iota(jnp.int32, (CH, CW), 0)
    col_b = jax.lax.broadcasted_iota(jnp.int32, (CH, CW), 1)
    dbase = col_b - row_b
    for rb in range(n_pos // CH):
        ws = min(max(rb * CH - CH, 0), n_pos - CW)
        shift = ws - rb * CH
        sub = jnp.zeros((CH, CW), jnp.float32)
        for i, (dr, dc) in enumerate(_OFFS):
            off = dr * height + dc
            sub = jnp.where(dbase == off - shift, rolled[i][:, ws:ws + CW], sub)
        att_ref[0, rb * CH:(rb + 1) * CH, ws:ws + CW] = sub


def kernel(x, Wq, bq, Wk, bk, Wv, bv, gamma):
    B, C, width, height = x.shape
    N = width * height
    d = Wq.shape[0]
    xf = x.reshape(B, C, N)

    import functools
    body = functools.partial(_body, width=width, height=height)
    out3, att = pl.pallas_call(
        body,
        grid=(B,),
        in_specs=[
            pl.BlockSpec((1, C, N), lambda b: (b, 0, 0)),
            pl.BlockSpec((d, C), lambda b: (0, 0)),
            pl.BlockSpec((d, 1), lambda b: (0, 0)),
            pl.BlockSpec((d, C), lambda b: (0, 0)),
            pl.BlockSpec((d, 1), lambda b: (0, 0)),
            pl.BlockSpec((C, C), lambda b: (0, 0)),
            pl.BlockSpec((C, 1), lambda b: (0, 0)),
            pl.BlockSpec((1, 1), lambda b: (0, 0)),
        ],
        out_specs=[
            pl.BlockSpec((1, C, N), lambda b: (b, 0, 0)),
            pl.BlockSpec((1, N, N), lambda b: (b, 0, 0)),
        ],
        out_shape=[
            jax.ShapeDtypeStruct((B, C, N), jnp.float32),
            jax.ShapeDtypeStruct((B, N, N), jnp.float32),
        ],
    )(xf, Wq, bq.reshape(d, 1), Wk, bk.reshape(d, 1), Wv,
      bv.reshape(C, 1), gamma.reshape(1, 1))
    return out3.reshape(B, C, width, height), att
